# baseline (speedup 1.0000x reference)
import functools

import jax
import jax.numpy as jnp
from jax import lax
from jax.experimental import pallas as pl
from jax.experimental.pallas import tpu as pltpu
from jax.experimental.pallas import tpu_sc as plsc

_GRP = 128
_L = 16


def _gather_body(b_per_w, tab_hbm, ids_hbm, out_hbm, ids_v, sems):
    wid = lax.axis_index("s") * 2 + lax.axis_index("c")
    base = wid * b_per_w
    pltpu.sync_copy(ids_hbm.at[pl.ds(base, b_per_w)], ids_v)

    n_batch = b_per_w // _L

    def fire(b):
        vec = ids_v[pl.ds(b * _L, _L)]
        for i in range(_L):
            r = vec[i]
            pltpu.make_async_copy(
                tab_hbm.at[pl.ds(r, 1), :],
                out_hbm.at[pl.ds(base + b * _L + i, 1), :],
                sems.at[0],
            ).start()

    def drain(b):
        for i in range(_L):
            pltpu.make_async_copy(
                tab_hbm.at[pl.ds(0, 1), :],
                out_hbm.at[pl.ds(base + b * _L + i, 1), :],
                sems.at[0],
            ).wait()

    fire(0)

    def per_batch(b, _):
        fire(b + 1)
        drain(b)
        return _

    lax.fori_loop(0, n_batch - 1, per_batch, 0, unroll=False)
    drain(n_batch - 1)


@jax.jit
def _embed_lookup(node_ids, node_embed_weight):
    b = node_ids.shape[0]
    info = plsc.get_sparse_core_info()
    nw = info.num_cores * info.num_subcores
    b_per_w = b // nw
    mesh = plsc.VectorSubcoreMesh(core_axis_name="c", subcore_axis_name="s")
    k = pl.kernel(
        functools.partial(_gather_body, b_per_w),
        mesh=mesh,
        out_type=jax.ShapeDtypeStruct((b, 64), jnp.float32),
        scratch_types=[
            pltpu.VMEM((b_per_w,), jnp.int32),
            pltpu.SemaphoreType.DMA((1,)),
        ],
    )
    return k(node_embed_weight, node_ids)


def kernel(node_ids, node_tids, type_ids, node_embed_weight):
    return _embed_lookup(node_ids.astype(jnp.int32), node_embed_weight)


# speedup vs baseline: 1.6386x; 1.6386x over previous
import functools

import jax
import jax.numpy as jnp
from jax import lax
from jax.experimental import pallas as pl
from jax.experimental.pallas import tpu as pltpu
from jax.experimental.pallas import tpu_sc as plsc

_GRP = 128
_L = 16


def _gather_body(b_per_w, tab_hbm, ids_hbm, out_hbm, ids_v, stage_v, sems):
    wid = lax.axis_index("s") * 2 + lax.axis_index("c")
    base = wid * b_per_w
    pltpu.sync_copy(ids_hbm.at[pl.ds(base, b_per_w)], ids_v)

    n_batch = _GRP // _L
    n_grp = b_per_w // _GRP

    def fire(g, sbuf, b):
        vec = ids_v[pl.ds(g * _GRP + b * _L, _L)]
        for i in range(_L):
            r = vec[i]
            pltpu.make_async_copy(
                tab_hbm.at[pl.ds(r, 1), :],
                stage_v.at[sbuf, pl.ds(b * _L + i, 1), :],
                sems.at[0],
            ).start()

    def drain(sbuf, b):
        for i in range(_L):
            pltpu.make_async_copy(
                tab_hbm.at[pl.ds(0, 1), :],
                stage_v.at[sbuf, pl.ds(b * _L + i, 1), :],
                sems.at[0],
            ).wait()

    for g in range(n_grp):
        sbuf = g % 2
        if g >= 2:
            pltpu.make_async_copy(
                stage_v.at[sbuf],
                out_hbm.at[pl.ds(base + (g - 2) * _GRP, _GRP)],
                sems.at[1],
            ).wait()

        fire(g, sbuf, 0)

        def per_batch(b, _):
            fire(g, sbuf, b + 1)
            drain(sbuf, b)
            return _

        lax.fori_loop(0, n_batch - 1, per_batch, 0, unroll=False)
        drain(sbuf, n_batch - 1)
        pltpu.make_async_copy(
            stage_v.at[sbuf],
            out_hbm.at[pl.ds(base + g * _GRP, _GRP)],
            sems.at[1],
        ).start()

    for g in range(n_grp - 2, n_grp):
        pltpu.make_async_copy(
            stage_v.at[g % 2],
            out_hbm.at[pl.ds(base + g * _GRP, _GRP)],
            sems.at[1],
        ).wait()


@jax.jit
def _embed_lookup(node_ids, node_embed_weight):
    b = node_ids.shape[0]
    info = plsc.get_sparse_core_info()
    nw = info.num_cores * info.num_subcores
    b_per_w = b // nw
    mesh = plsc.VectorSubcoreMesh(core_axis_name="c", subcore_axis_name="s")
    k = pl.kernel(
        functools.partial(_gather_body, b_per_w),
        mesh=mesh,
        out_type=jax.ShapeDtypeStruct((b, 64), jnp.float32),
        scratch_types=[
            pltpu.VMEM((b_per_w,), jnp.int32),
            pltpu.VMEM((2, _GRP, 64), jnp.float32),
            pltpu.SemaphoreType.DMA((2,)),
        ],
    )
    return k(node_embed_weight, node_ids)


def kernel(node_ids, node_tids, type_ids, node_embed_weight):
    return _embed_lookup(node_ids.astype(jnp.int32), node_embed_weight)
